# baseline (speedup 1.0000x reference)
import functools

import jax
import jax.numpy as jnp
from jax import lax
from jax.experimental import pallas as pl
from jax.experimental.pallas import tpu as pltpu
from jax.experimental.pallas import tpu_sc as plsc

B = 4096
POOL = 1024
D = 1024
KSEL = 5
PLEN = 5
ROW = PLEN * D

TOPK_BLK = 512

NC = 2
NS = 16
NW = NC * NS
TOTAL_IDX = B * KSEL
B_PER_W = TOTAL_IDX // NW
GCHUNK = 8
N_CHUNKS = B_PER_W // GCHUNK


def _topk_body(x_ref, k_ref, idx_ref, loss_ref):
    i = pl.program_id(0)
    xb = x_ref[...]
    keys = k_ref[...]
    dot = lax.dot_general(xb, keys, (((1,), (1,)), ((), ())),
                          preferred_element_type=jnp.float32)
    xn = jnp.sqrt(jnp.sum(xb * xb, axis=1, keepdims=True))
    kn = jnp.sqrt(jnp.sum(keys * keys, axis=1, keepdims=True))
    denom = jnp.maximum(xn * kn.reshape(1, POOL), 1e-8)
    dist = 1.0 - dot / denom

    iota = lax.broadcasted_iota(jnp.int32, dist.shape, 1)
    cur = dist
    idx_cols = []
    total = jnp.zeros((), jnp.float32)
    for _ in range(KSEL):
        m = jnp.min(cur, axis=1, keepdims=True)
        am = jnp.min(jnp.where(cur == m, iota, POOL), axis=1,
                     keepdims=True)
        idx_cols.append(am)
        total = total + jnp.sum(m)
        cur = jnp.where(iota == am, jnp.inf, cur)
    idx_ref[...] = jnp.concatenate(idx_cols, axis=1)

    @pl.when(i == 0)
    def _():
        loss_ref[...] = jnp.zeros((1, 1), jnp.float32)

    loss_ref[...] += total.reshape(1, 1)


def _topk_select(xq, keys):
    grid = (B // TOPK_BLK,)
    idx, loss_sum = pl.pallas_call(
        _topk_body,
        grid=grid,
        in_specs=[
            pl.BlockSpec((TOPK_BLK, D), lambda i: (i, 0)),
            pl.BlockSpec((POOL, D), lambda i: (0, 0)),
        ],
        out_specs=[
            pl.BlockSpec((TOPK_BLK, KSEL), lambda i: (i, 0)),
            pl.BlockSpec((1, 1), lambda i: (0, 0)),
        ],
        out_shape=[
            jax.ShapeDtypeStruct((B, KSEL), jnp.int32),
            jax.ShapeDtypeStruct((1, 1), jnp.float32),
        ],
    )(xq, keys)
    return idx, loss_sum


def _sc_gather(values_flat, idx_flat):
    mesh = plsc.VectorSubcoreMesh(core_axis_name="c", subcore_axis_name="s")

    @functools.partial(
        pl.kernel,
        mesh=mesh,
        out_type=jax.ShapeDtypeStruct((TOTAL_IDX, ROW), jnp.float32),
        scratch_types=[
            pltpu.VMEM((B_PER_W,), jnp.int32),
            pltpu.VMEM((GCHUNK, ROW), jnp.float32),
            pltpu.VMEM((GCHUNK, ROW), jnp.float32),
            pltpu.SemaphoreType.DMA,
            pltpu.SemaphoreType.DMA,
        ],
    )
    def gk(values_hbm, idx_hbm, out_hbm, idx_v, buf0, buf1, sem0, sem1):
        wid = lax.axis_index("s") * NC + lax.axis_index("c")
        base = wid * B_PER_W
        pltpu.sync_copy(idx_hbm.at[pl.ds(base, B_PER_W)], idx_v)

        @pl.loop(0, N_CHUNKS // 2)
        def _(t):
            j0 = 2 * t * GCHUNK
            j1 = (2 * t + 1) * GCHUNK
            cp0 = pltpu.async_copy(
                values_hbm.at[idx_v.at[pl.ds(j0, GCHUNK)]], buf0, sem0)
            cp1 = pltpu.async_copy(
                values_hbm.at[idx_v.at[pl.ds(j1, GCHUNK)]], buf1, sem1)
            cp0.wait()
            pltpu.sync_copy(buf0, out_hbm.at[pl.ds(base + j0, GCHUNK)])
            cp1.wait()
            pltpu.sync_copy(buf1, out_hbm.at[pl.ds(base + j1, GCHUNK)])

    return gk(values_flat, idx_flat)


GBP = 256
SUB = 8
NRS = SUB * PLEN


def _tc_gather_body(idx_ref, v_ref, out_ref, s_ref):
    ri = lax.broadcasted_iota(jnp.int32, (NRS, NRS), 0)
    ci = lax.broadcasted_iota(jnp.int32, (NRS, NRS), 1)
    src = (ri % SUB) * PLEN + ri // SUB
    perm = jnp.where(ci == src, 1.0, 0.0)
    k = pl.program_id(0)
    for g in range(GBP // SUB):
        for b in range(SUB):
            j = idx_ref[g * SUB + b, k]
            s_ref[b * PLEN:(b + 1) * PLEN, :] = v_ref[j]
        t = lax.dot_general(perm, s_ref[...], (((1,), (0,)), ((), ())),
                            preferred_element_type=jnp.float32)
        out_ref[:, g * SUB:(g + 1) * SUB, :] = t.reshape(PLEN, SUB, D)


def _tc_gather_t(values, idx):
    return pl.pallas_call(
        _tc_gather_body,
        grid=(KSEL, B // GBP),
        in_specs=[
            pl.BlockSpec((GBP, KSEL), lambda k, i: (i, 0),
                         memory_space=pltpu.SMEM),
            pl.BlockSpec((POOL, PLEN, D), lambda k, i: (0, 0, 0)),
        ],
        out_specs=pl.BlockSpec((PLEN, GBP, D), lambda k, i: (k, i, 0)),
        out_shape=jax.ShapeDtypeStruct((KSEL * PLEN, B, D), jnp.float32),
        scratch_shapes=[pltpu.VMEM((NRS, D), jnp.float32)],
    )(idx, values)


def kernel(x, keys, values):
    xq = x[:, 0, :]
    idx, loss_sum = _topk_select(xq, keys)
    key_loss = loss_sum[0, 0] / (B * KSEL)
    out_t = _tc_gather_t(values, idx)
    out = jnp.transpose(out_t, (1, 0, 2))
    return (out, key_loss)


# speedup vs baseline: 1.1119x; 1.1119x over previous
import functools

import jax
import jax.numpy as jnp
from jax import lax
from jax.experimental import pallas as pl
from jax.experimental.pallas import tpu as pltpu
from jax.experimental.pallas import tpu_sc as plsc

B = 4096
POOL = 1024
D = 1024
KSEL = 5
PLEN = 5
ROW = PLEN * D

TOPK_BLK = 512

NC = 2
NS = 16
NW = NC * NS
TOTAL_IDX = B * KSEL
B_PER_W = TOTAL_IDX // NW
GCHUNK = 8
N_CHUNKS = B_PER_W // GCHUNK


def _topk_body(x_ref, k_ref, idx_ref, loss_ref):
    i = pl.program_id(0)
    xb = x_ref[...]
    keys = k_ref[...]
    dot = lax.dot_general(xb, keys, (((1,), (1,)), ((), ())),
                          preferred_element_type=jnp.float32)
    xn = jnp.sqrt(jnp.sum(xb * xb, axis=1, keepdims=True))
    kn = jnp.sqrt(jnp.sum(keys * keys, axis=1, keepdims=True))
    denom = jnp.maximum(xn * kn.reshape(1, POOL), 1e-8)
    dist = 1.0 - dot / denom

    iota = lax.broadcasted_iota(jnp.int32, dist.shape, 1)
    cur = dist
    idx_cols = []
    total = jnp.zeros((), jnp.float32)
    for _ in range(KSEL):
        m = jnp.min(cur, axis=1, keepdims=True)
        am = jnp.min(jnp.where(cur == m, iota, POOL), axis=1,
                     keepdims=True)
        idx_cols.append(am)
        total = total + jnp.sum(m)
        cur = jnp.where(iota == am, jnp.inf, cur)
    idx_ref[...] = jnp.concatenate(idx_cols, axis=1)

    @pl.when(i == 0)
    def _():
        loss_ref[...] = jnp.zeros((1, 1), jnp.float32)

    loss_ref[...] += total.reshape(1, 1)


def _topk_select(xq, keys):
    grid = (B // TOPK_BLK,)
    idx, loss_sum = pl.pallas_call(
        _topk_body,
        grid=grid,
        in_specs=[
            pl.BlockSpec((TOPK_BLK, D), lambda i: (i, 0)),
            pl.BlockSpec((POOL, D), lambda i: (0, 0)),
        ],
        out_specs=[
            pl.BlockSpec((TOPK_BLK, KSEL), lambda i: (i, 0)),
            pl.BlockSpec((1, 1), lambda i: (0, 0)),
        ],
        out_shape=[
            jax.ShapeDtypeStruct((B, KSEL), jnp.int32),
            jax.ShapeDtypeStruct((1, 1), jnp.float32),
        ],
    )(xq, keys)
    return idx, loss_sum


def _sc_gather(values_flat, idx_flat):
    mesh = plsc.VectorSubcoreMesh(core_axis_name="c", subcore_axis_name="s")

    @functools.partial(
        pl.kernel,
        mesh=mesh,
        out_type=jax.ShapeDtypeStruct((TOTAL_IDX, ROW), jnp.float32),
        scratch_types=[
            pltpu.VMEM((B_PER_W,), jnp.int32),
            pltpu.VMEM((GCHUNK, ROW), jnp.float32),
            pltpu.VMEM((GCHUNK, ROW), jnp.float32),
            pltpu.SemaphoreType.DMA,
            pltpu.SemaphoreType.DMA,
        ],
    )
    def gk(values_hbm, idx_hbm, out_hbm, idx_v, buf0, buf1, sem0, sem1):
        wid = lax.axis_index("s") * NC + lax.axis_index("c")
        base = wid * B_PER_W
        pltpu.sync_copy(idx_hbm.at[pl.ds(base, B_PER_W)], idx_v)

        @pl.loop(0, N_CHUNKS // 2)
        def _(t):
            j0 = 2 * t * GCHUNK
            j1 = (2 * t + 1) * GCHUNK
            cp0 = pltpu.async_copy(
                values_hbm.at[idx_v.at[pl.ds(j0, GCHUNK)]], buf0, sem0)
            cp1 = pltpu.async_copy(
                values_hbm.at[idx_v.at[pl.ds(j1, GCHUNK)]], buf1, sem1)
            cp0.wait()
            pltpu.sync_copy(buf0, out_hbm.at[pl.ds(base + j0, GCHUNK)])
            cp1.wait()
            pltpu.sync_copy(buf1, out_hbm.at[pl.ds(base + j1, GCHUNK)])

    return gk(values_flat, idx_flat)


GB = 64
SUB = 8
NR = SUB * KSEL * PLEN


def _tc_gather_body(idx_ref, v_ref, out_ref, s_ref):
    ri = lax.broadcasted_iota(jnp.int32, (NR, NR), 0)
    ci = lax.broadcasted_iota(jnp.int32, (NR, NR), 1)
    src = (ri % SUB) * (KSEL * PLEN) + ri // SUB
    perm = jnp.where(ci == src, 1.0, 0.0)
    for g in range(GB // SUB):
        for b in range(SUB):
            for k in range(KSEL):
                j = idx_ref[g * SUB + b, k]
                base = b * KSEL * PLEN + k * PLEN
                s_ref[base:base + PLEN, :] = v_ref[j]
        t = lax.dot_general(perm, s_ref[...], (((1,), (0,)), ((), ())),
                            preferred_element_type=jnp.float32)
        out_ref[:, g * SUB:(g + 1) * SUB, :] = t.reshape(KSEL * PLEN, SUB, D)


def _tc_gather_t(values, idx):
    return pl.pallas_call(
        _tc_gather_body,
        grid=(B // GB,),
        in_specs=[
            pl.BlockSpec((GB, KSEL), lambda i: (i, 0),
                         memory_space=pltpu.SMEM),
            pl.BlockSpec((POOL, PLEN, D), lambda i: (0, 0, 0)),
        ],
        out_specs=pl.BlockSpec((KSEL * PLEN, GB, D), lambda i: (0, i, 0)),
        out_shape=jax.ShapeDtypeStruct((KSEL * PLEN, B, D), jnp.float32),
        scratch_shapes=[pltpu.VMEM((NR, D), jnp.float32)],
    )(idx, values)


def kernel(x, keys, values):
    xq = x[:, 0, :]
    idx, loss_sum = _topk_select(xq, keys)
    key_loss = loss_sum[0, 0] / (B * KSEL)
    out_t = _tc_gather_t(values, idx)
    out = jnp.transpose(out_t, (1, 0, 2))
    return (out, key_loss)
